# baseline (speedup 1.0000x reference)
import jax
import jax.numpy as jnp
from jax import lax
from jax.experimental import pallas as pl
from jax.experimental.pallas import tpu as pltpu
from jax.experimental.pallas import tpu_sc as plsc

N = 10000
NP = 10240
E = 320000
D_FEAT = 128
HIDDEN = 64
OUT = 64
HEADS = 8
DH = 8
MAX_PATH = 3

NC = 2
NS = 16
ROW = 80
CHUNK = 100
EDGES_PER_TILE = E // (NC * NS)
CHUNKS_PER_TILE = EDGES_PER_TILE // CHUNK
ZROWS = 64
ROWS_PER_TILE = NP // NS


def _pre_body(x_ref, wemb_ref, bemb_ref, asrc_ref, adst_ref, p64_ref,
              cures_ref, ed_ref, bvec_ref, edmax_ref):
    h0 = jnp.dot(x_ref[...], wemb_ref[...],
                 preferred_element_type=jnp.float32) + bemb_ref[...]
    h0t = jnp.dot(h0, p64_ref[...], preferred_element_type=jnp.float32)
    es = jnp.dot(h0, asrc_ref[...], preferred_element_type=jnp.float32)
    ed = jnp.dot(h0, adst_ref[...], preferred_element_type=jnp.float32)
    padrow = jnp.zeros((NP - N, ROW), jnp.float32)
    cures_ref[...] = jnp.concatenate(
        [jnp.concatenate([h0t, es, es], axis=1), padrow], axis=0)
    ed_ref[...] = jnp.concatenate(
        [jnp.concatenate([ed, ed], axis=1), padrow[:, :16]], axis=0)
    esmax = jnp.max(es, axis=0, keepdims=True)
    edmax = jnp.max(ed, axis=0, keepdims=True)
    t = esmax + edmax
    b = jnp.maximum(t, 0.2 * t)
    z1 = jnp.zeros((1, 8), jnp.float32)
    bvec_ref[...] = jnp.concatenate([b, b], axis=1)
    edmax_ref[...] = jnp.concatenate([edmax, z1], axis=1)


_pre_call = pl.pallas_call(
    _pre_body,
    out_shape=(
        jax.ShapeDtypeStruct((NP, ROW), jnp.float32),
        jax.ShapeDtypeStruct((NP, 16), jnp.float32),
        jax.ShapeDtypeStruct((1, 16), jnp.float32),
        jax.ShapeDtypeStruct((1, 16), jnp.float32),
    ),
)


NBUF = 3


def _sc_body(cures_hbm, ed_hbm, src_hbm, dst_hbm, bvec_hbm, part_hbm,
             idx_s, idx_d, *bufs):
    c = lax.axis_index("c")
    s = lax.axis_index("s")
    srcb = bufs[0:NBUF]
    dstb = bufs[NBUF:2 * NBUF]
    outb = bufs[2 * NBUF:3 * NBUF]
    bvec_v, zbuf, acc = bufs[3 * NBUF:3 * NBUF + 3]
    sem_s = bufs[3 * NBUF + 3:4 * NBUF + 3]
    sem_d = bufs[4 * NBUF + 3:5 * NBUF + 3]
    sem_c = bufs[5 * NBUF + 3:6 * NBUF + 3]

    def _zero(i, _):
        zline = jnp.zeros((16,), jnp.float32)
        for k in range(ROW // 16):
            zbuf[i, pl.ds(16 * k, 16)] = zline
        return 0
    lax.fori_loop(0, ZROWS, _zero, 0)
    rowbase = s * ROWS_PER_TILE
    for k in range(ROWS_PER_TILE // ZROWS):
        pltpu.sync_copy(zbuf, acc.at[pl.ds(rowbase + k * ZROWS, ZROWS)])
    plsc.subcore_barrier()

    wid = c * NS + s
    pltpu.sync_copy(src_hbm.at[wid], idx_s)
    pltpu.sync_copy(dst_hbm.at[wid], idx_d)
    pltpu.sync_copy(bvec_hbm, bvec_v)
    bv = bvec_v[...]

    def _start_gather(g, b):
        pltpu.async_copy(cures_hbm.at[idx_s.at[g]], srcb[b], sem_s[b])
        pltpu.async_copy(ed_hbm.at[idx_d.at[g]], dstb[b], sem_d[b])

    def _wait_gather(g, b):
        pltpu.make_async_copy(cures_hbm.at[idx_s.at[g]], srcb[b],
                              sem_s[b]).wait()
        pltpu.make_async_copy(ed_hbm.at[idx_d.at[g]], dstb[b],
                              sem_d[b]).wait()

    def _start_scatter(g, b):
        pltpu.async_copy(outb[b], acc.at[idx_d.at[g]], sem_c[b], add=True)

    def _wait_scatter(g, b):
        pltpu.make_async_copy(outb[b], acc.at[idx_d.at[g]], sem_c[b]).wait()

    def _compute(b):
        src_r, dst_r, out_r = srcb[b], dstb[b], outb[b]

        @plsc.parallel_loop(0, CHUNK, step=1, unroll=4)
        def _edge(j):
            es16 = src_r[j, pl.ds(64, 16)]
            ed16 = dst_r[j, pl.ds(0, 16)]
            e = es16 + ed16
            e = jnp.maximum(e, 0.2 * e)
            ex = jnp.exp(e - bv)
            out_r[j, pl.ds(64, 16)] = ex
            for k in range(4):
                out_r[j, pl.ds(16 * k, 16)] = (
                    src_r[j, pl.ds(16 * k, 16)] * ex)

    for b in range(NBUF - 1):
        _start_gather(b, b)

    def _group(p, _):
        for b in range(NBUF):
            g = NBUF * p + b
            ga = g + NBUF - 1
            bg = (NBUF - 1 + b) % NBUF

            @pl.when(ga < CHUNKS_PER_TILE)
            def _():
                _start_gather(ga, bg)
            _wait_gather(g, b)

            @pl.when(g >= NBUF)
            def _():
                _wait_scatter(g - NBUF, b)
            _compute(b)
            _start_scatter(g, b)
        return 0
    ngroups = CHUNKS_PER_TILE // NBUF
    lax.fori_loop(0, ngroups, _group, 0)
    for g in range(ngroups * NBUF, CHUNKS_PER_TILE):
        b = g % NBUF
        _wait_gather(g, b)
        if g >= NBUF:
            _wait_scatter(g - NBUF, b)
        _compute(b)
        _start_scatter(g, b)
    for g in range(CHUNKS_PER_TILE - NBUF, CHUNKS_PER_TILE):
        _wait_scatter(g, g % NBUF)

    plsc.subcore_barrier()
    for k in range(ROWS_PER_TILE // ZROWS):
        r0 = rowbase + k * ZROWS
        pltpu.sync_copy(acc.at[pl.ds(r0, ZROWS)], zbuf)
        pltpu.sync_copy(zbuf, part_hbm.at[c, pl.ds(r0, ZROWS)])


_sc_call = pl.kernel(
    _sc_body,
    out_type=jax.ShapeDtypeStruct((NC, NP, ROW), jnp.float32),
    mesh=plsc.VectorSubcoreMesh(core_axis_name="c", subcore_axis_name="s"),
    compiler_params=pltpu.CompilerParams(use_tc_tiling_on_sc=False),
    scratch_types=(
        [
            pltpu.VMEM((CHUNKS_PER_TILE, CHUNK), jnp.int32),
            pltpu.VMEM((CHUNKS_PER_TILE, CHUNK), jnp.int32),
        ]
        + [pltpu.VMEM((CHUNK, ROW), jnp.float32)] * NBUF
        + [pltpu.VMEM((CHUNK, 16), jnp.float32)] * NBUF
        + [pltpu.VMEM((CHUNK, ROW), jnp.float32)] * NBUF
        + [
            pltpu.VMEM((16,), jnp.float32),
            pltpu.VMEM((ZROWS, ROW), jnp.float32),
            pltpu.VMEM_SHARED((NP, ROW), jnp.float32),
        ]
        + [pltpu.SemaphoreType.DMA] * (3 * NBUF)
    ),
)


def _node_body(part_ref, edmax_ref, asrc_ref, bselt_ref, p64_ref, wmeta_ref,
               bmeta_ref, qmeta_ref, wimp_ref, cures_ref, bvec_ref,
               stats_ref):
    acc = part_ref[0] + part_ref[1]
    sb = jnp.dot(acc, bselt_ref[...],
                 preferred_element_type=jnp.float32)
    agg = acc[:, :64] / (sb + 1e-16)
    pe_t = jnp.where(agg > 0, agg,
                     jnp.exp(jnp.minimum(agg, 0.0)) - 1.0)
    pe = jnp.dot(pe_t, p64_ref[...], preferred_element_type=jnp.float32)
    es = jnp.dot(pe, asrc_ref[...], preferred_element_type=jnp.float32)
    cures_ref[...] = jnp.concatenate([pe_t, es, es], axis=1)
    esmax = jnp.max(es, axis=0, keepdims=True)
    t = esmax + edmax_ref[...][:, :8]
    b = jnp.maximum(t, 0.2 * t)
    bvec_ref[...] = jnp.concatenate([b, b], axis=1)
    pe = pe[:N]
    th = jnp.tanh(jnp.dot(pe, wmeta_ref[...],
                          preferred_element_type=jnp.float32) + bmeta_ref[...])
    s_r = jnp.dot(jnp.sum(th, axis=0, keepdims=True) / N, qmeta_ref[...],
                  preferred_element_type=jnp.float32)
    pw = jnp.dot(jnp.sum(pe, axis=0, keepdims=True) / N, wimp_ref[...],
                 preferred_element_type=jnp.float32)
    stats_ref[...] = jnp.concatenate([s_r, pw], axis=1)


_node_call = pl.pallas_call(
    _node_body,
    out_shape=(
        jax.ShapeDtypeStruct((NP, ROW), jnp.float32),
        jax.ShapeDtypeStruct((1, 16), jnp.float32),
        jax.ShapeDtypeStruct((1, 2), jnp.float32),
    ),
)


def _final_body(p1_ref, p2_ref, p3_ref, stats_ref, p64_ref, wout_ref,
                bout_ref, out_ref):
    logits = stats_ref[...][:, 0:1] + stats_ref[...][:, 1:2]
    m = jnp.max(logits, axis=0, keepdims=True)
    eb = jnp.exp(logits - m)
    beta = eb / jnp.sum(eb, axis=0, keepdims=True)
    final = (beta[0:1, 0:1] * p1_ref[...][:N, :64]
             + beta[1:2, 0:1] * p2_ref[...][:N, :64]
             + beta[2:3, 0:1] * p3_ref[...][:N, :64])
    wout_p = jnp.dot(p64_ref[...], wout_ref[...],
                     preferred_element_type=jnp.float32)
    out_ref[...] = jnp.dot(final, wout_p,
                           preferred_element_type=jnp.float32) + bout_ref[...]


_final_call = pl.pallas_call(
    _final_body,
    out_shape=jax.ShapeDtypeStruct((N, OUT), jnp.float32),
)


def kernel(x, edge_index, W_emb, b_emb, a_src, a_dst, w_imp, W_meta, b_meta,
           q_meta, W_out, b_out):
    eye = jnp.eye(HEADS, dtype=jnp.float32)
    asrc_m = (a_src[:, :, None] * eye[:, None, :]).reshape(HIDDEN, HEADS)
    adst_m = (a_dst[:, :, None] * eye[:, None, :]).reshape(HIDDEN, HEADS)
    permv = jnp.array([(l % DH) * DH + l // DH for l in range(HIDDEN)])
    p64 = jnp.eye(HIDDEN, dtype=jnp.float32)[permv]
    bselt = jnp.concatenate(
        [jnp.zeros((HIDDEN, HIDDEN), jnp.float32),
         jnp.tile(eye, (1, DH)),
         jnp.zeros((8, HIDDEN), jnp.float32)], axis=0)

    src2d = edge_index[0].reshape(NC * NS, CHUNKS_PER_TILE, CHUNK)
    dst2d = edge_index[1].reshape(NC * NS, CHUNKS_PER_TILE, CHUNK)

    cures, ed16, bvec, edmax = _pre_call(
        x, W_emb, b_emb.reshape(1, HIDDEN), asrc_m, adst_m, p64)

    stats_list = []
    pes = []
    for _ in range(MAX_PATH):
        part = _sc_call(cures, ed16, src2d, dst2d, bvec.reshape(16))
        cures, bvec, st = _node_call(
            part, edmax, asrc_m, bselt, p64, W_meta,
            b_meta.reshape(1, HIDDEN), q_meta.reshape(HIDDEN, 1),
            w_imp.reshape(HIDDEN, 1))
        stats_list.append(st)
        pes.append(cures)

    stats = jnp.concatenate(stats_list, axis=0)
    return _final_call(pes[0], pes[1], pes[2], stats, p64, W_out,
                       b_out.reshape(1, OUT))


# speedup vs baseline: 1.0273x; 1.0273x over previous
import jax
import jax.numpy as jnp
from jax import lax
from jax.experimental import pallas as pl
from jax.experimental.pallas import tpu as pltpu
from jax.experimental.pallas import tpu_sc as plsc

N = 10000
NP = 10240
E = 320000
D_FEAT = 128
HIDDEN = 64
OUT = 64
HEADS = 8
DH = 8
MAX_PATH = 3

NC = 2
NS = 16
ROW = 80
CHUNK = 100
EDGES_PER_TILE = E // (NC * NS)
CHUNKS_PER_TILE = EDGES_PER_TILE // CHUNK
ZROWS = 64
ROWS_PER_TILE = NP // NS


def _pre_body(x_ref, wemb_ref, bemb_ref, asrc_ref, adst_ref, p64_ref,
              cures_ref, ed_ref, bvec_ref, edmax_ref):
    h0 = jnp.dot(x_ref[...], wemb_ref[...],
                 preferred_element_type=jnp.float32) + bemb_ref[...]
    h0t = jnp.dot(h0, p64_ref[...], preferred_element_type=jnp.float32)
    es = jnp.dot(h0, asrc_ref[...], preferred_element_type=jnp.float32)
    ed = jnp.dot(h0, adst_ref[...], preferred_element_type=jnp.float32)
    padrow = jnp.zeros((NP - N, ROW), jnp.float32)
    cures_ref[...] = jnp.concatenate(
        [jnp.concatenate([h0t, es, es], axis=1), padrow], axis=0)
    ed_ref[...] = jnp.concatenate(
        [jnp.concatenate([ed, ed], axis=1), padrow[:, :16]], axis=0)
    esmax = jnp.max(es, axis=0, keepdims=True)
    edmax = jnp.max(ed, axis=0, keepdims=True)
    t = esmax + edmax
    b = jnp.maximum(t, 0.2 * t)
    z1 = jnp.zeros((1, 8), jnp.float32)
    bvec_ref[...] = jnp.concatenate([b, b], axis=1)
    edmax_ref[...] = jnp.concatenate([edmax, z1], axis=1)


_pre_call = pl.pallas_call(
    _pre_body,
    out_shape=(
        jax.ShapeDtypeStruct((NP, ROW), jnp.float32),
        jax.ShapeDtypeStruct((NP, 16), jnp.float32),
        jax.ShapeDtypeStruct((1, 16), jnp.float32),
        jax.ShapeDtypeStruct((1, 16), jnp.float32),
    ),
)


NBUF = 3


def _sc_body(cures_hbm, ed_hbm, src_hbm, dst_hbm, bvec_hbm, part_hbm,
             idx_s, idx_d, *bufs):
    c = lax.axis_index("c")
    s = lax.axis_index("s")
    srcb = bufs[0:NBUF]
    dstb = bufs[NBUF:2 * NBUF]
    outb = bufs[2 * NBUF:3 * NBUF]
    bvec_v, zbuf, acc = bufs[3 * NBUF:3 * NBUF + 3]
    sem_s = bufs[3 * NBUF + 3:4 * NBUF + 3]
    sem_d = bufs[4 * NBUF + 3:5 * NBUF + 3]
    sem_c = bufs[5 * NBUF + 3:6 * NBUF + 3]

    def _zero(i, _):
        zline = jnp.zeros((16,), jnp.float32)
        for k in range(ROW // 16):
            zbuf[i, pl.ds(16 * k, 16)] = zline
        return 0
    lax.fori_loop(0, ZROWS, _zero, 0)
    rowbase = s * ROWS_PER_TILE
    for k in range(ROWS_PER_TILE // ZROWS):
        pltpu.sync_copy(zbuf, acc.at[pl.ds(rowbase + k * ZROWS, ZROWS)])
    plsc.subcore_barrier()

    wid = c * NS + s
    pltpu.sync_copy(src_hbm.at[wid], idx_s)
    pltpu.sync_copy(dst_hbm.at[wid], idx_d)
    pltpu.sync_copy(bvec_hbm, bvec_v)
    bv = bvec_v[...]

    def _start_gather(g, b):
        pltpu.async_copy(cures_hbm.at[idx_s.at[g]], srcb[b], sem_s[b])
        pltpu.async_copy(ed_hbm.at[idx_d.at[g]], dstb[b], sem_d[b])

    def _wait_gather(g, b):
        pltpu.make_async_copy(cures_hbm.at[idx_s.at[g]], srcb[b],
                              sem_s[b]).wait()
        pltpu.make_async_copy(ed_hbm.at[idx_d.at[g]], dstb[b],
                              sem_d[b]).wait()

    def _start_scatter(g, b):
        pltpu.async_copy(outb[b], acc.at[idx_d.at[g]], sem_c[b], add=True)

    def _wait_scatter(g, b):
        pltpu.make_async_copy(outb[b], acc.at[idx_d.at[g]], sem_c[b]).wait()

    def _compute(b):
        src_r, dst_r, out_r = srcb[b], dstb[b], outb[b]

        @plsc.parallel_loop(0, CHUNK, step=1, unroll=4)
        def _edge(j):
            es16 = src_r[j, pl.ds(64, 16)]
            ed16 = dst_r[j, pl.ds(0, 16)]
            e = es16 + ed16
            e = jnp.maximum(e, 0.2 * e)
            ex = jnp.exp(e - bv)
            out_r[j, pl.ds(64, 16)] = ex
            for k in range(4):
                out_r[j, pl.ds(16 * k, 16)] = (
                    src_r[j, pl.ds(16 * k, 16)] * ex)

    for b in range(NBUF - 1):
        _start_gather(b, b)

    def _group(p, _):
        for b in range(NBUF):
            g = NBUF * p + b
            ga = g + NBUF - 1
            bg = (NBUF - 1 + b) % NBUF

            @pl.when(ga < CHUNKS_PER_TILE)
            def _():
                _start_gather(ga, bg)
            _wait_gather(g, b)

            @pl.when(g >= NBUF)
            def _():
                _wait_scatter(g - NBUF, b)
            _compute(b)
            _start_scatter(g, b)
        return 0
    ngroups = CHUNKS_PER_TILE // NBUF
    lax.fori_loop(0, ngroups, _group, 0)
    for g in range(ngroups * NBUF, CHUNKS_PER_TILE):
        b = g % NBUF
        _wait_gather(g, b)
        if g >= NBUF:
            _wait_scatter(g - NBUF, b)
        _compute(b)
        _start_scatter(g, b)
    for g in range(CHUNKS_PER_TILE - NBUF, CHUNKS_PER_TILE):
        _wait_scatter(g, g % NBUF)

    plsc.subcore_barrier()
    pltpu.sync_copy(acc.at[pl.ds(rowbase, ROWS_PER_TILE)],
                    part_hbm.at[c, pl.ds(rowbase, ROWS_PER_TILE)])


_sc_call = pl.kernel(
    _sc_body,
    out_type=jax.ShapeDtypeStruct((NC, NP, ROW), jnp.float32),
    mesh=plsc.VectorSubcoreMesh(core_axis_name="c", subcore_axis_name="s"),
    compiler_params=pltpu.CompilerParams(use_tc_tiling_on_sc=False),
    scratch_types=(
        [
            pltpu.VMEM((CHUNKS_PER_TILE, CHUNK), jnp.int32),
            pltpu.VMEM((CHUNKS_PER_TILE, CHUNK), jnp.int32),
        ]
        + [pltpu.VMEM((CHUNK, ROW), jnp.float32)] * NBUF
        + [pltpu.VMEM((CHUNK, 16), jnp.float32)] * NBUF
        + [pltpu.VMEM((CHUNK, ROW), jnp.float32)] * NBUF
        + [
            pltpu.VMEM((16,), jnp.float32),
            pltpu.VMEM((ZROWS, ROW), jnp.float32),
            pltpu.VMEM_SHARED((NP, ROW), jnp.float32),
        ]
        + [pltpu.SemaphoreType.DMA] * (3 * NBUF)
    ),
)


def _node_body(part_ref, edmax_ref, asrc_ref, bselt_ref, p64_ref, wmeta_ref,
               bmeta_ref, qmeta_ref, wimp_ref, cures_ref, bvec_ref,
               stats_ref):
    acc = part_ref[0] + part_ref[1]
    sb = jnp.dot(acc, bselt_ref[...],
                 preferred_element_type=jnp.float32)
    agg = acc[:, :64] / (sb + 1e-16)
    pe_t = jnp.where(agg > 0, agg,
                     jnp.exp(jnp.minimum(agg, 0.0)) - 1.0)
    pe = jnp.dot(pe_t, p64_ref[...], preferred_element_type=jnp.float32)
    es = jnp.dot(pe, asrc_ref[...], preferred_element_type=jnp.float32)
    cures_ref[...] = jnp.concatenate([pe_t, es, es], axis=1)
    esmax = jnp.max(es, axis=0, keepdims=True)
    t = esmax + edmax_ref[...][:, :8]
    b = jnp.maximum(t, 0.2 * t)
    bvec_ref[...] = jnp.concatenate([b, b], axis=1)
    pe = pe[:N]
    th = jnp.tanh(jnp.dot(pe, wmeta_ref[...],
                          preferred_element_type=jnp.float32) + bmeta_ref[...])
    s_r = jnp.dot(jnp.sum(th, axis=0, keepdims=True) / N, qmeta_ref[...],
                  preferred_element_type=jnp.float32)
    pw = jnp.dot(jnp.sum(pe, axis=0, keepdims=True) / N, wimp_ref[...],
                 preferred_element_type=jnp.float32)
    stats_ref[...] = jnp.concatenate([s_r, pw], axis=1)


_node_call = pl.pallas_call(
    _node_body,
    out_shape=(
        jax.ShapeDtypeStruct((NP, ROW), jnp.float32),
        jax.ShapeDtypeStruct((1, 16), jnp.float32),
        jax.ShapeDtypeStruct((1, 2), jnp.float32),
    ),
)


def _node_final_body(part_ref, bselt_ref, p64_ref, wmeta_ref, bmeta_ref,
                     qmeta_ref, wimp_ref, s12_ref, p1_ref, p2_ref, wout_ref,
                     bout_ref, out_ref):
    acc = part_ref[0] + part_ref[1]
    sb = jnp.dot(acc, bselt_ref[...],
                 preferred_element_type=jnp.float32)
    agg = acc[:, :64] / (sb + 1e-16)
    pe_t = jnp.where(agg > 0, agg,
                     jnp.exp(jnp.minimum(agg, 0.0)) - 1.0)
    pe_t = pe_t[:N]
    pe = jnp.dot(pe_t, p64_ref[...], preferred_element_type=jnp.float32)
    th = jnp.tanh(jnp.dot(pe, wmeta_ref[...],
                          preferred_element_type=jnp.float32) + bmeta_ref[...])
    s_r = jnp.dot(jnp.sum(th, axis=0, keepdims=True) / N, qmeta_ref[...],
                  preferred_element_type=jnp.float32)
    pw = jnp.dot(jnp.sum(pe, axis=0, keepdims=True) / N, wimp_ref[...],
                 preferred_element_type=jnp.float32)
    s12 = s12_ref[...]
    logits = jnp.concatenate(
        [s12[:, 0:1] + s12[:, 1:2], s_r + pw], axis=0)
    m = jnp.max(logits, axis=0, keepdims=True)
    eb = jnp.exp(logits - m)
    beta = eb / jnp.sum(eb, axis=0, keepdims=True)
    final = (beta[0:1, 0:1] * p1_ref[...][:N, :64]
             + beta[1:2, 0:1] * p2_ref[...][:N, :64]
             + beta[2:3, 0:1] * pe_t)
    wout_p = jnp.dot(p64_ref[...], wout_ref[...],
                     preferred_element_type=jnp.float32)
    out_ref[...] = jnp.dot(final, wout_p,
                           preferred_element_type=jnp.float32) + bout_ref[...]


_node_final_call = pl.pallas_call(
    _node_final_body,
    out_shape=jax.ShapeDtypeStruct((N, OUT), jnp.float32),
)


def kernel(x, edge_index, W_emb, b_emb, a_src, a_dst, w_imp, W_meta, b_meta,
           q_meta, W_out, b_out):
    eye = jnp.eye(HEADS, dtype=jnp.float32)
    asrc_m = (a_src[:, :, None] * eye[:, None, :]).reshape(HIDDEN, HEADS)
    adst_m = (a_dst[:, :, None] * eye[:, None, :]).reshape(HIDDEN, HEADS)
    permv = jnp.array([(l % DH) * DH + l // DH for l in range(HIDDEN)])
    p64 = jnp.eye(HIDDEN, dtype=jnp.float32)[permv]
    bselt = jnp.concatenate(
        [jnp.zeros((HIDDEN, HIDDEN), jnp.float32),
         jnp.tile(eye, (1, DH)),
         jnp.zeros((8, HIDDEN), jnp.float32)], axis=0)

    src2d = edge_index[0].reshape(NC * NS, CHUNKS_PER_TILE, CHUNK)
    dst2d = edge_index[1].reshape(NC * NS, CHUNKS_PER_TILE, CHUNK)

    cures, ed16, bvec, edmax = _pre_call(
        x, W_emb, b_emb.reshape(1, HIDDEN), asrc_m, adst_m, p64)

    stats_list = []
    pes = []
    for _ in range(MAX_PATH - 1):
        part = _sc_call(cures, ed16, src2d, dst2d, bvec.reshape(16))
        cures, bvec, st = _node_call(
            part, edmax, asrc_m, bselt, p64, W_meta,
            b_meta.reshape(1, HIDDEN), q_meta.reshape(HIDDEN, 1),
            w_imp.reshape(HIDDEN, 1))
        stats_list.append(st)
        pes.append(cures)

    part = _sc_call(cures, ed16, src2d, dst2d, bvec.reshape(16))
    stats12 = jnp.concatenate(stats_list, axis=0)
    return _node_final_call(
        part, bselt, p64, W_meta, b_meta.reshape(1, HIDDEN),
        q_meta.reshape(HIDDEN, 1), w_imp.reshape(HIDDEN, 1), stats12,
        pes[0], pes[1], W_out, b_out.reshape(1, OUT))


# speedup vs baseline: 1.1256x; 1.0956x over previous
import jax
import jax.numpy as jnp
from jax import lax
from jax.experimental import pallas as pl
from jax.experimental.pallas import tpu as pltpu
from jax.experimental.pallas import tpu_sc as plsc

N = 10000
NP = 10240
E = 320000
D_FEAT = 128
HIDDEN = 64
OUT = 64
HEADS = 8
DH = 8
MAX_PATH = 3

NC = 2
NS = 16
ROW = 80
CROW = 64
CHUNK = 100
EDGES_PER_TILE = E // (NC * NS)
CHUNKS_PER_TILE = EDGES_PER_TILE // CHUNK
ZROWS = 64
ROWS_PER_TILE = NP // NS


def _pre_body(x_ref, wemb_ref, bemb_ref, asrc_ref, adst_ref, p64_ref,
              cures_ref, ed_ref, bvec_ref, edmax_ref):
    h0 = jnp.dot(x_ref[...], wemb_ref[...],
                 preferred_element_type=jnp.float32) + bemb_ref[...]
    h0t = jnp.dot(h0, p64_ref[...], preferred_element_type=jnp.float32)
    es = jnp.dot(h0, asrc_ref[...], preferred_element_type=jnp.float32)
    ed = jnp.dot(h0, adst_ref[...], preferred_element_type=jnp.float32)
    padrow = jnp.zeros((NP - N, CROW), jnp.float32)
    cures_ref[...] = jnp.concatenate([h0t, padrow], axis=0)
    ed_ref[...] = jnp.concatenate(
        [jnp.concatenate([ed, ed], axis=1), padrow[:, :16]], axis=0)
    esmax = jnp.max(es, axis=0, keepdims=True)
    edmax = jnp.max(ed, axis=0, keepdims=True)
    t = esmax + edmax
    b = jnp.maximum(t, 0.2 * t)
    z1 = jnp.zeros((1, 8), jnp.float32)
    bvec_ref[...] = jnp.concatenate([b, b], axis=1)
    edmax_ref[...] = jnp.concatenate([edmax, z1], axis=1)


_pre_call = pl.pallas_call(
    _pre_body,
    out_shape=(
        jax.ShapeDtypeStruct((NP, CROW), jnp.float32),
        jax.ShapeDtypeStruct((NP, 16), jnp.float32),
        jax.ShapeDtypeStruct((1, 16), jnp.float32),
        jax.ShapeDtypeStruct((1, 16), jnp.float32),
    ),
)


NBUF = 3


def _sc_body(cures_hbm, ed_hbm, src_hbm, dst_hbm, bvec_hbm, asrct_hbm,
             part_hbm, idx_s, idx_d, *bufs):
    c = lax.axis_index("c")
    s = lax.axis_index("s")
    srcb = bufs[0:NBUF]
    dstb = bufs[NBUF:2 * NBUF]
    outb = bufs[2 * NBUF:3 * NBUF]
    bvec_v, asrct_v, zbuf, acc = bufs[3 * NBUF:3 * NBUF + 4]
    sem_s = bufs[3 * NBUF + 4:4 * NBUF + 4]
    sem_d = bufs[4 * NBUF + 4:5 * NBUF + 4]
    sem_c = bufs[5 * NBUF + 4:6 * NBUF + 4]

    def _zero(i, _):
        zline = jnp.zeros((16,), jnp.float32)
        for k in range(ROW // 16):
            zbuf[i, pl.ds(16 * k, 16)] = zline
        return 0
    lax.fori_loop(0, ZROWS, _zero, 0)
    rowbase = s * ROWS_PER_TILE
    for k in range(ROWS_PER_TILE // ZROWS):
        pltpu.sync_copy(zbuf, acc.at[pl.ds(rowbase + k * ZROWS, ZROWS)])
    plsc.subcore_barrier()

    wid = c * NS + s
    pltpu.sync_copy(src_hbm.at[wid], idx_s)
    pltpu.sync_copy(dst_hbm.at[wid], idx_d)
    pltpu.sync_copy(bvec_hbm, bvec_v)
    pltpu.sync_copy(asrct_hbm, asrct_v)
    bv = bvec_v[...]
    a_t = [asrct_v[pl.ds(16 * k, 16)] for k in range(4)]
    swapidx = (lax.iota(jnp.int32, 16) + 8) % 16
    _gdn = lax.GatherDimensionNumbers(
        offset_dims=(), collapsed_slice_dims=(0,), start_index_map=(0,))

    def _halfswap(v):
        return lax.gather(v, swapidx[:, None], dimension_numbers=_gdn,
                          slice_sizes=(1,),
                          mode=lax.GatherScatterMode.PROMISE_IN_BOUNDS)

    def _start_gather(g, b):
        pltpu.async_copy(cures_hbm.at[idx_s.at[g]], srcb[b], sem_s[b])
        pltpu.async_copy(ed_hbm.at[idx_d.at[g]], dstb[b], sem_d[b])

    def _wait_gather(g, b):
        pltpu.make_async_copy(cures_hbm.at[idx_s.at[g]], srcb[b],
                              sem_s[b]).wait()
        pltpu.make_async_copy(ed_hbm.at[idx_d.at[g]], dstb[b],
                              sem_d[b]).wait()

    def _start_scatter(g, b):
        pltpu.async_copy(outb[b], acc.at[idx_d.at[g]], sem_c[b], add=True)

    def _wait_scatter(g, b):
        pltpu.make_async_copy(outb[b], acc.at[idx_d.at[g]], sem_c[b]).wait()

    def _compute(b):
        src_r, dst_r, out_r = srcb[b], dstb[b], outb[b]

        @plsc.parallel_loop(0, CHUNK, step=1, unroll=4)
        def _edge(j):
            cv = [src_r[j, pl.ds(16 * k, 16)] for k in range(4)]
            pp = (cv[0] * a_t[0] + cv[1] * a_t[1]
                  + cv[2] * a_t[2] + cv[3] * a_t[3])
            es16 = pp + _halfswap(pp)
            ed16 = dst_r[j, pl.ds(0, 16)]
            e = es16 + ed16
            e = jnp.maximum(e, 0.2 * e)
            ex = jnp.exp(e - bv)
            out_r[j, pl.ds(64, 16)] = ex
            for k in range(4):
                out_r[j, pl.ds(16 * k, 16)] = cv[k] * ex

    for b in range(NBUF - 1):
        _start_gather(b, b)

    def _group(p, _):
        for b in range(NBUF):
            g = NBUF * p + b
            ga = g + NBUF - 1
            bg = (NBUF - 1 + b) % NBUF

            @pl.when(ga < CHUNKS_PER_TILE)
            def _():
                _start_gather(ga, bg)
            _wait_gather(g, b)

            @pl.when(g >= NBUF)
            def _():
                _wait_scatter(g - NBUF, b)
            _compute(b)
            _start_scatter(g, b)
        return 0
    ngroups = CHUNKS_PER_TILE // NBUF
    lax.fori_loop(0, ngroups, _group, 0)
    for g in range(ngroups * NBUF, CHUNKS_PER_TILE):
        b = g % NBUF
        _wait_gather(g, b)
        if g >= NBUF:
            _wait_scatter(g - NBUF, b)
        _compute(b)
        _start_scatter(g, b)
    for g in range(CHUNKS_PER_TILE - NBUF, CHUNKS_PER_TILE):
        _wait_scatter(g, g % NBUF)

    plsc.subcore_barrier()
    pltpu.sync_copy(acc.at[pl.ds(rowbase, ROWS_PER_TILE)],
                    part_hbm.at[c, pl.ds(rowbase, ROWS_PER_TILE)])


_sc_call = pl.kernel(
    _sc_body,
    out_type=jax.ShapeDtypeStruct((NC, NP, ROW), jnp.float32),
    mesh=plsc.VectorSubcoreMesh(core_axis_name="c", subcore_axis_name="s"),
    compiler_params=pltpu.CompilerParams(use_tc_tiling_on_sc=False),
    scratch_types=(
        [
            pltpu.VMEM((CHUNKS_PER_TILE, CHUNK), jnp.int32),
            pltpu.VMEM((CHUNKS_PER_TILE, CHUNK), jnp.int32),
        ]
        + [pltpu.VMEM((CHUNK, CROW), jnp.float32)] * NBUF
        + [pltpu.VMEM((CHUNK, 16), jnp.float32)] * NBUF
        + [pltpu.VMEM((CHUNK, ROW), jnp.float32)] * NBUF
        + [
            pltpu.VMEM((16,), jnp.float32),
            pltpu.VMEM((64,), jnp.float32),
            pltpu.VMEM((ZROWS, ROW), jnp.float32),
            pltpu.VMEM_SHARED((NP, ROW), jnp.float32),
        ]
        + [pltpu.SemaphoreType.DMA] * (3 * NBUF)
    ),
)


def _node_body(part_ref, edmax_ref, asrc_ref, bselt_ref, p64_ref, wmeta_ref,
               bmeta_ref, qmeta_ref, wimp_ref, cures_ref, bvec_ref,
               stats_ref):
    acc = part_ref[0] + part_ref[1]
    sb = jnp.dot(acc, bselt_ref[...],
                 preferred_element_type=jnp.float32)
    agg = acc[:, :64] / (sb + 1e-16)
    pe_t = jnp.where(agg > 0, agg,
                     jnp.exp(jnp.minimum(agg, 0.0)) - 1.0)
    pe = jnp.dot(pe_t, p64_ref[...], preferred_element_type=jnp.float32)
    es = jnp.dot(pe, asrc_ref[...], preferred_element_type=jnp.float32)
    cures_ref[...] = pe_t
    esmax = jnp.max(es, axis=0, keepdims=True)
    t = esmax + edmax_ref[...][:, :8]
    b = jnp.maximum(t, 0.2 * t)
    bvec_ref[...] = jnp.concatenate([b, b], axis=1)
    pe = pe[:N]
    th = jnp.tanh(jnp.dot(pe, wmeta_ref[...],
                          preferred_element_type=jnp.float32) + bmeta_ref[...])
    s_r = jnp.dot(jnp.sum(th, axis=0, keepdims=True) / N, qmeta_ref[...],
                  preferred_element_type=jnp.float32)
    pw = jnp.dot(jnp.sum(pe, axis=0, keepdims=True) / N, wimp_ref[...],
                 preferred_element_type=jnp.float32)
    stats_ref[...] = jnp.concatenate([s_r, pw], axis=1)


_node_call = pl.pallas_call(
    _node_body,
    out_shape=(
        jax.ShapeDtypeStruct((NP, CROW), jnp.float32),
        jax.ShapeDtypeStruct((1, 16), jnp.float32),
        jax.ShapeDtypeStruct((1, 2), jnp.float32),
    ),
)


def _node_final_body(part_ref, bselt_ref, p64_ref, wmeta_ref, bmeta_ref,
                     qmeta_ref, wimp_ref, s12_ref, p1_ref, p2_ref, wout_ref,
                     bout_ref, out_ref):
    acc = part_ref[0] + part_ref[1]
    sb = jnp.dot(acc, bselt_ref[...],
                 preferred_element_type=jnp.float32)
    agg = acc[:, :64] / (sb + 1e-16)
    pe_t = jnp.where(agg > 0, agg,
                     jnp.exp(jnp.minimum(agg, 0.0)) - 1.0)
    pe_t = pe_t[:N]
    pe = jnp.dot(pe_t, p64_ref[...], preferred_element_type=jnp.float32)
    th = jnp.tanh(jnp.dot(pe, wmeta_ref[...],
                          preferred_element_type=jnp.float32) + bmeta_ref[...])
    s_r = jnp.dot(jnp.sum(th, axis=0, keepdims=True) / N, qmeta_ref[...],
                  preferred_element_type=jnp.float32)
    pw = jnp.dot(jnp.sum(pe, axis=0, keepdims=True) / N, wimp_ref[...],
                 preferred_element_type=jnp.float32)
    s12 = s12_ref[...]
    logits = jnp.concatenate(
        [s12[:, 0:1] + s12[:, 1:2], s_r + pw], axis=0)
    m = jnp.max(logits, axis=0, keepdims=True)
    eb = jnp.exp(logits - m)
    beta = eb / jnp.sum(eb, axis=0, keepdims=True)
    final = (beta[0:1, 0:1] * p1_ref[...][:N, :64]
             + beta[1:2, 0:1] * p2_ref[...][:N, :64]
             + beta[2:3, 0:1] * pe_t)
    wout_p = jnp.dot(p64_ref[...], wout_ref[...],
                     preferred_element_type=jnp.float32)
    out_ref[...] = jnp.dot(final, wout_p,
                           preferred_element_type=jnp.float32) + bout_ref[...]


_node_final_call = pl.pallas_call(
    _node_final_body,
    out_shape=jax.ShapeDtypeStruct((N, OUT), jnp.float32),
)


def kernel(x, edge_index, W_emb, b_emb, a_src, a_dst, w_imp, W_meta, b_meta,
           q_meta, W_out, b_out):
    eye = jnp.eye(HEADS, dtype=jnp.float32)
    asrc_m = (a_src[:, :, None] * eye[:, None, :]).reshape(HIDDEN, HEADS)
    adst_m = (a_dst[:, :, None] * eye[:, None, :]).reshape(HIDDEN, HEADS)
    permv = jnp.array([(l % DH) * DH + l // DH for l in range(HIDDEN)])
    p64 = jnp.eye(HIDDEN, dtype=jnp.float32)[permv]
    bselt = jnp.concatenate(
        [jnp.zeros((HIDDEN, HIDDEN), jnp.float32),
         jnp.tile(eye, (1, DH)),
         jnp.zeros((8, HIDDEN), jnp.float32)], axis=0)

    at_idx = jnp.array([(f % 8) * DH + 2 * (f // 16) + (f % 16) // 8
                        for f in range(HIDDEN)])
    asrc_t = a_src.reshape(-1)[at_idx]

    src2d = edge_index[0].reshape(NC * NS, CHUNKS_PER_TILE, CHUNK)
    dst2d = edge_index[1].reshape(NC * NS, CHUNKS_PER_TILE, CHUNK)

    cures, ed16, bvec, edmax = _pre_call(
        x, W_emb, b_emb.reshape(1, HIDDEN), asrc_m, adst_m, p64)

    stats_list = []
    pes = []
    for _ in range(MAX_PATH - 1):
        part = _sc_call(cures, ed16, src2d, dst2d, bvec.reshape(16), asrc_t)
        cures, bvec, st = _node_call(
            part, edmax, asrc_m, bselt, p64, W_meta,
            b_meta.reshape(1, HIDDEN), q_meta.reshape(HIDDEN, 1),
            w_imp.reshape(HIDDEN, 1))
        stats_list.append(st)
        pes.append(cures)

    part = _sc_call(cures, ed16, src2d, dst2d, bvec.reshape(16), asrc_t)
    stats12 = jnp.concatenate(stats_list, axis=0)
    return _node_final_call(
        part, bselt, p64, W_meta, b_meta.reshape(1, HIDDEN),
        q_meta.reshape(HIDDEN, 1), w_imp.reshape(HIDDEN, 1), stats12,
        pes[0], pes[1], W_out, b_out.reshape(1, OUT))
